# Initial kernel scaffold; baseline (speedup 1.0000x reference)
import jax
import jax.numpy as jnp
from jax.experimental import pallas as pl


def kernel(x_thuoc, x_benh, edge_index, W_thuoc, b_thuoc, W_benh, b_benh, sigma, W1, b1, W2, b2, W3, b3):
    raise NotImplementedError("write your pallas kernel here")


# speedup vs baseline: 13.0924x; 13.0924x over previous
import functools

import jax
import jax.numpy as jnp
from jax import lax
from jax.experimental import pallas as pl
from jax.experimental.pallas import tpu as pltpu
from jax.experimental.pallas import tpu_sc as plsc

NT = 16
NCORE = 2
CHUNK = 128
SUP = 4
TRASH_PAD = 48
BN = 1000


def _make_sc_agg(n_nodes, n_tile_chunks, width):
  n_sup = n_tile_chunks // SUP
  acc_rows = n_nodes + TRASH_PAD
  rz = acc_rows // NT
  mesh = plsc.VectorSubcoreMesh(
      core_axis_name="c", subcore_axis_name="s",
      num_cores=NCORE, num_subcores=NT)

  @functools.partial(
      pl.kernel, mesh=mesh,
      out_type=jax.ShapeDtypeStruct((NCORE * acc_rows, width), jnp.float32),
      scratch_types=[
          pltpu.VMEM_SHARED((acc_rows, width), jnp.float32),
          pltpu.VMEM((SUP, CHUNK), jnp.int32),
          pltpu.VMEM((SUP, CHUNK), jnp.int32),
          pltpu.VMEM((SUP, CHUNK), jnp.int32),
          pltpu.VMEM((SUP * CHUNK, width), jnp.float32),
          pltpu.SemaphoreType.DMA,
      ],
      compiler_params=pltpu.CompilerParams(use_tc_tiling_on_sc=False),
  )
  def agg(rows_hbm, src_hbm, dst_hbm, zeros_hbm, out_hbm,
          acc, srcb, sadj, dstb, rows, sem):
    c = lax.axis_index("c")
    s = lax.axis_index("s")
    pltpu.sync_copy(zeros_hbm, acc.at[pl.ds(s * rz, rz)])
    plsc.subcore_barrier()
    off = c * n_nodes

    @pl.loop(0, n_sup)
    def _(g):
      pltpu.sync_copy(src_hbm.at[s, pl.ds(g * SUP, SUP)], srcb)
      pltpu.sync_copy(dst_hbm.at[s, pl.ds(g * SUP, SUP)], dstb)
      for j in range(SUP):
        for i in range(CHUNK // 16):
          sadj[j, pl.ds(i * 16, 16)] = srcb[j, pl.ds(i * 16, 16)] + off
      descs = [
          pltpu.async_copy(rows_hbm.at[sadj.at[j]],
                           rows.at[pl.ds(j * CHUNK, CHUNK)], sem)
          for j in range(SUP)
      ]
      for d in descs:
        d.wait()
      for j in range(SUP):
        pltpu.sync_copy(rows.at[pl.ds(j * CHUNK, CHUNK)],
                        acc.at[dstb.at[j]], add=True)

    plsc.subcore_barrier()
    pltpu.sync_copy(acc.at[pl.ds(s * rz, rz)],
                    out_hbm.at[pl.ds(c * acc_rows + s * rz, rz)])

  return agg


def _make_sc_deg(n_nodes, n_chunks_per_worker):
  acc_rows = n_nodes + TRASH_PAD
  rz = acc_rows // NT
  mesh = plsc.VectorSubcoreMesh(
      core_axis_name="c", subcore_axis_name="s",
      num_cores=NCORE, num_subcores=NT)

  @functools.partial(
      pl.kernel, mesh=mesh,
      out_type=jax.ShapeDtypeStruct((NCORE, acc_rows, 16), jnp.float32),
      scratch_types=[
          pltpu.VMEM_SHARED((acc_rows, 16), jnp.float32),
          pltpu.VMEM((n_chunks_per_worker, CHUNK), jnp.int32),
          pltpu.VMEM((CHUNK, 16), jnp.float32),
      ],
      compiler_params=pltpu.CompilerParams(use_tc_tiling_on_sc=False),
  )
  def deg(dst_hbm, ones_hbm, zeros_hbm, out_hbm, accd, dstb, onesb):
    c = lax.axis_index("c")
    s = lax.axis_index("s")
    wid = c * NT + s
    pltpu.sync_copy(zeros_hbm, accd.at[pl.ds(s * rz, rz)])
    pltpu.sync_copy(ones_hbm, onesb)
    pltpu.sync_copy(dst_hbm.at[wid], dstb)
    plsc.subcore_barrier()
    for j in range(n_chunks_per_worker):
      pltpu.sync_copy(onesb, accd.at[dstb.at[j]], add=True)
    plsc.subcore_barrier()
    pltpu.sync_copy(accd.at[pl.ds(s * rz, rz)], out_hbm.at[c, pl.ds(s * rz, rz)])

  return deg


def _enc_body(x_ref, w_ref, b_ref, sig_ref, o_ref):
  h = jnp.dot(x_ref[...], w_ref[0], preferred_element_type=jnp.float32)
  h = h + b_ref[0]
  sig = sig_ref[0, 0]
  o_ref[...] = jnp.exp(-(h * h) * (0.5 / (sig * sig)))


def _encode(x_all, w_stack, b_stack, sigma, n_half):
  nb = n_half // BN
  return pl.pallas_call(
      _enc_body,
      grid=(2, nb),
      in_specs=[
          pl.BlockSpec((BN, x_all.shape[1]), lambda t, i: (t * nb + i, 0)),
          pl.BlockSpec((1,) + w_stack.shape[1:], lambda t, i: (t, 0, 0)),
          pl.BlockSpec((1, 1, b_stack.shape[2]), lambda t, i: (t, 0, 0)),
          pl.BlockSpec(memory_space=pltpu.SMEM),
      ],
      out_specs=pl.BlockSpec((BN, w_stack.shape[2]), lambda t, i: (t * nb + i, 0)),
      out_shape=jax.ShapeDtypeStruct((2 * n_half, w_stack.shape[2]), jnp.float32),
  )(x_all, w_stack, b_stack, sigma)


def _first_body(h_ref, w_ref, dp_ref, xwp_ref, dinv_ref):
  dinv = lax.rsqrt(1.0 + dp_ref[0] + dp_ref[1])
  xw = jnp.dot(h_ref[...], w_ref[...], preferred_element_type=jnp.float32)
  xw = xw * dinv
  half = xw.shape[1] // 2
  xwp_ref[0] = xw[:, :half]
  xwp_ref[1] = xw[:, half:]
  dinv_ref[...] = dinv


def _first_layer(h, w, dp, n):
  nb = n // BN
  d = w.shape[1]
  return pl.pallas_call(
      _first_body,
      grid=(nb,),
      in_specs=[
          pl.BlockSpec((BN, h.shape[1]), lambda i: (i, 0)),
          pl.BlockSpec(w.shape, lambda i: (0, 0)),
          pl.BlockSpec((2, BN, 1), lambda i: (0, i, 0)),
      ],
      out_specs=[
          pl.BlockSpec((2, BN, d // 2), lambda i: (0, i, 0)),
          pl.BlockSpec((BN, 1), lambda i: (i, 0)),
      ],
      out_shape=[
          jax.ShapeDtypeStruct((2, n, d // 2), jnp.float32),
          jax.ShapeDtypeStruct((n, 1), jnp.float32),
      ],
  )(h, w, dp)


def _combine_body(acc_ref, xwp_ref, dinv_ref, b_ref, w_ref, o_ref):
  agg = jnp.concatenate(
      [acc_ref[0] + xwp_ref[0], acc_ref[1] + xwp_ref[1]], axis=1)
  dinv = dinv_ref[...]
  h = jnp.maximum(dinv * agg + b_ref[...], 0.0)
  y = jnp.dot(h, w_ref[...], preferred_element_type=jnp.float32) * dinv
  half = y.shape[1] // 2
  o_ref[0] = y[:, :half]
  o_ref[1] = y[:, half:]


def _combine(acc, xwp, dinv, b, w, n):
  nb = n // BN
  hw = acc.shape[2]
  d = w.shape[1]
  return pl.pallas_call(
      _combine_body,
      grid=(nb,),
      in_specs=[
          pl.BlockSpec((2, BN, hw), lambda i: (0, i, 0)),
          pl.BlockSpec((2, BN, hw), lambda i: (0, i, 0)),
          pl.BlockSpec((BN, 1), lambda i: (i, 0)),
          pl.BlockSpec((1, 2 * hw), lambda i: (0, 0)),
          pl.BlockSpec(w.shape, lambda i: (0, 0)),
      ],
      out_specs=pl.BlockSpec((2, BN, d // 2), lambda i: (0, i, 0)),
      out_shape=jax.ShapeDtypeStruct((2, n, d // 2), jnp.float32),
  )(acc, xwp, dinv, b, w)


def _final_body(acc_ref, xwp_ref, dinv_ref, b_ref, o_ref):
  agg = jnp.concatenate(
      [acc_ref[0] + xwp_ref[0], acc_ref[1] + xwp_ref[1]], axis=1)
  o_ref[...] = dinv_ref[...] * agg + b_ref[...]


def _final(acc, xwp, dinv, b, n):
  nb = n // BN
  hw = acc.shape[2]
  return pl.pallas_call(
      _final_body,
      grid=(nb,),
      in_specs=[
          pl.BlockSpec((2, BN, hw), lambda i: (0, i, 0)),
          pl.BlockSpec((2, BN, hw), lambda i: (0, i, 0)),
          pl.BlockSpec((BN, 1), lambda i: (i, 0)),
          pl.BlockSpec((1, 2 * hw), lambda i: (0, 0)),
      ],
      out_specs=pl.BlockSpec((BN, 2 * hw), lambda i: (i, 0)),
      out_shape=jax.ShapeDtypeStruct((n, 2 * hw), jnp.float32),
  )(acc, xwp, dinv, b)


def kernel(x_thuoc, x_benh, edge_index, W_thuoc, b_thuoc, W_benh, b_benh,
           sigma, W1, b1, W2, b2, W3, b3):
  n_half = x_thuoc.shape[0]
  n = n_half + x_benh.shape[0]
  e = edge_index.shape[1]

  unit = NT * CHUNK * SUP
  e_pad = -(-e // unit) * unit
  pad = e_pad - e
  src = edge_index[0]
  dst = edge_index[1]
  src_p = jnp.concatenate([src, jnp.zeros((pad,), jnp.int32)])
  dst_p = jnp.concatenate([dst, jnp.full((pad,), n, jnp.int32)])
  src_r = src_p.reshape(NT, -1, CHUNK)
  dst_r = dst_p.reshape(NT, -1, CHUNK)
  dst_deg = dst_p.reshape(NT * NCORE, -1, CHUNK)
  n_tile_chunks = src_r.shape[1]
  n_deg_chunks = dst_deg.shape[1]

  acc_rows = n + TRASH_PAD
  z16 = jnp.zeros((acc_rows // NT, 16), jnp.float32)
  z32 = jnp.zeros((acc_rows // NT, 32), jnp.float32)
  ones16 = jnp.ones((CHUNK, 16), jnp.float32)

  x_all = jnp.concatenate([x_thuoc, x_benh], axis=0)
  w_stack = jnp.stack([W_thuoc, W_benh])
  b_stack = jnp.stack([b_thuoc, b_benh])[:, None, :]
  sig = jnp.reshape(sigma, (1, 1))

  sc_agg32 = _make_sc_agg(n, n_tile_chunks, 32)
  sc_agg16 = _make_sc_agg(n, n_tile_chunks, 16)
  sc_deg = _make_sc_deg(n, n_deg_chunks)

  h = _encode(x_all, w_stack, b_stack, sig, n_half)
  degp = sc_deg(dst_deg, ones16, z16)
  dp = degp[:, :n, 0:1]

  def unpad(a, w):
    return a.reshape(2, acc_rows, w)[:, :n]

  xwp1, dinv = _first_layer(h, W1, dp, n)
  acc1 = unpad(sc_agg32(xwp1.reshape(2 * n, 32), src_r, dst_r, z32), 32)
  xwp2 = _combine(acc1, xwp1, dinv, b1[None, :], W2, n)
  acc2 = unpad(sc_agg32(xwp2.reshape(2 * n, 32), src_r, dst_r, z32), 32)
  xwp3 = _combine(acc2, xwp2, dinv, b2[None, :], W3, n)
  acc3 = unpad(sc_agg16(xwp3.reshape(2 * n, 16), src_r, dst_r, z16), 16)
  return _final(acc3, xwp3, dinv, b3[None, :], n)
